# baseline (speedup 1.0000x reference)
import functools

import jax
import jax.numpy as jnp
from jax import lax
from jax.experimental import pallas as pl
from jax.experimental.pallas import tpu as pltpu
from jax.experimental.pallas import tpu_sc as plsc

_NC, _NS = 2, 16
_R = 16
_B_SC = 3


def _tc_add(emb_ref, pos_ref, out_ref):
    out_ref[...] = emb_ref[...] + pos_ref[...]


def _sc_batch3(embeddings, pos_table):
    B, S, D = embeddings.shape
    nw = _NC * _NS
    s_per_w = S // nw
    n_chunks = s_per_w // _R
    lanes_per_row = D // 16

    mesh = plsc.VectorSubcoreMesh(
        core_axis_name="c", subcore_axis_name="s", num_cores=_NC, num_subcores=_NS
    )

    @functools.partial(
        pl.kernel,
        out_type=jax.ShapeDtypeStruct((1, S, D), jnp.float32),
        mesh=mesh,
        scratch_types=[
            [pltpu.VMEM((_R, D), jnp.float32) for _ in range(2)],
            [pltpu.VMEM((_R, D), jnp.float32) for _ in range(2)],
            [pltpu.SemaphoreType.DMA for _ in range(2)],
            [pltpu.SemaphoreType.DMA for _ in range(2)],
            [pltpu.SemaphoreType.DMA for _ in range(2)],
        ],
    )
    def sc_add(emb_hbm, pos_hbm, out_hbm, pbufs, ebufs, psems, isems, osems):
        wid = lax.axis_index("s") * _NC + lax.axis_index("c")
        s_base = wid * s_per_w

        def start_chunk_in(i, bank):
            s0 = s_base + i * _R
            pltpu.async_copy(pos_hbm.at[pl.ds(s0, _R)], pbufs[bank], psems[bank])
            pltpu.async_copy(
                emb_hbm.at[_B_SC, pl.ds(s0, _R)], ebufs[bank], isems[bank]
            )

        start_chunk_in(0, 0)

        def pair(i2, carry):
            for bank in range(2):
                i = 2 * i2 + bank
                s0 = s_base + i * _R
                other = 1 - bank

                @pl.when(i + 1 < n_chunks)
                def _(i=i, bank=bank, other=other):
                    s_prev = s_base + (i - 1) * _R

                    @pl.when(i >= 1)
                    def _():
                        pltpu.make_async_copy(
                            ebufs[other],
                            out_hbm.at[0, pl.ds(s_prev, _R)],
                            osems[other],
                        ).wait()

                    start_chunk_in(i + 1, other)

                pltpu.make_async_copy(
                    pos_hbm.at[pl.ds(s0, _R)], pbufs[bank], psems[bank]
                ).wait()
                pltpu.make_async_copy(
                    emb_hbm.at[_B_SC, pl.ds(s0, _R)], ebufs[bank], isems[bank]
                ).wait()

                def add_row(r, carry2, bank=bank):
                    for j in range(lanes_per_row):
                        plsc.addupdate(
                            ebufs[bank].at[r, pl.ds(j * 16, 16)],
                            pbufs[bank][r, pl.ds(j * 16, 16)],
                        )
                    return carry2

                lax.fori_loop(0, _R, add_row, 0)
                pltpu.async_copy(
                    ebufs[bank], out_hbm.at[0, pl.ds(s0, _R)], osems[bank]
                )
            return carry

        lax.fori_loop(0, n_chunks // 2, pair, 0)

        for bank in range(2):
            i_last = n_chunks - 2 + bank
            s_last = s_base + i_last * _R
            pltpu.make_async_copy(
                ebufs[bank], out_hbm.at[0, pl.ds(s_last, _R)], osems[bank]
            ).wait()

    return sc_add(embeddings, pos_table)


def kernel(embeddings, pos_table):
    B, S, D = embeddings.shape
    BS = 2048
    out_sc = _sc_batch3(embeddings, pos_table)
    out_tc = pl.pallas_call(
        _tc_add,
        grid=(S // BS, _B_SC),
        in_specs=[
            pl.BlockSpec((1, BS, D), lambda s, b: (b, s, 0)),
            pl.BlockSpec((BS, D), lambda s, b: (s, 0)),
        ],
        out_specs=pl.BlockSpec((1, BS, D), lambda s, b: (b, s, 0)),
        out_shape=jax.ShapeDtypeStruct((_B_SC, S, D), embeddings.dtype),
    )(embeddings, pos_table)
    return jnp.concatenate([out_tc, out_sc], axis=0)
